# baseline (speedup 1.0000x reference)
import jax
import jax.numpy as jnp
from jax import lax
from jax.experimental import pallas as pl
from jax.experimental.pallas import tpu as pltpu
from jax.experimental.pallas import tpu_sc as plsc

D = 64
KNEG = 20
NC, NS = 2, 16
NW = NC * NS
ROWS = 24
GSZ = 16


def _fire_group(g, out_hbm, idx_o, idx_n, vo_buf, vng_buf, sem):
    col0 = pl.multiple_of(g * GSZ, 8)
    nbase = pl.multiple_of(g * (GSZ * KNEG), 8)
    pltpu.async_copy(out_hbm.at[idx_o.at[pl.ds(col0, GSZ)]], vo_buf, sem)
    pltpu.async_copy(out_hbm.at[idx_n.at[pl.ds(nbase, 128)]],
                     vng_buf.at[pl.ds(0, 128), :], sem)
    pltpu.async_copy(out_hbm.at[idx_n.at[pl.ds(nbase + 128, 128)]],
                     vng_buf.at[pl.ds(128, 128), :], sem)
    pltpu.async_copy(out_hbm.at[idx_n.at[pl.ds(nbase + 256, 64)]],
                     vng_buf.at[pl.ds(256, 64), :], sem)


def _wait_group(out_hbm, vo_buf, vng_buf, sem):
    pltpu.make_async_copy(out_hbm.at[pl.ds(0, GSZ)], vo_buf, sem).wait()
    pltpu.make_async_copy(out_hbm.at[pl.ds(0, 128)],
                          vng_buf.at[pl.ds(0, 128), :], sem).wait()
    pltpu.make_async_copy(out_hbm.at[pl.ds(0, 128)],
                          vng_buf.at[pl.ds(128, 128), :], sem).wait()
    pltpu.make_async_copy(out_hbm.at[pl.ds(0, 64)],
                          vng_buf.at[pl.ds(256, 64), :], sem).wait()


def _sc_scores_body(in_hbm, out_hbm, cen_hbm, ctx_hbm, neg_hbm, scores_hbm,
                    idx_c, idx_o, idx_n, vc_rows, vo_g, vng, scores_v,
                    sem_c, sem_a):
    bpw = vc_rows.shape[0]
    ng = bpw // GSZ
    wid = lax.axis_index("s") * NC + lax.axis_index("c")
    base = wid * bpw

    pltpu.sync_copy(cen_hbm.at[pl.ds(base, bpw)], idx_c)
    pltpu.sync_copy(ctx_hbm.at[pl.ds(base, bpw)], idx_o)
    pltpu.sync_copy(neg_hbm.at[pl.ds(base * KNEG, bpw * KNEG)], idx_n)

    cps = [pltpu.async_copy(in_hbm.at[idx_c.at[pl.ds(c * 128, 128)]],
                            vc_rows.at[pl.ds(c * 128, 128), :], sem_c)
           for c in range(bpw // 128)]
    _fire_group(0, out_hbm, idx_o, idx_n, vo_g.at[0], vng.at[0], sem_a.at[0])
    _fire_group(1, out_hbm, idx_o, idx_n, vo_g.at[1], vng.at[1], sem_a.at[1])
    for cp in cps:
        cp.wait()

    iota = lax.iota(jnp.int32, 16)
    iota_k = iota * KNEG
    big = jnp.full((16,), 1e4, jnp.float32)

    def _compute_group(g, vo_buf, vng_buf):
        col0 = pl.multiple_of(g * GSZ, 8)
        vc_idx = col0 + iota
        for k_lo, k_hi, with_pos in ((0, 6, True), (6, 13, False),
                                     (13, 20, False)):
            nacc = (k_hi - k_lo) + (1 if with_pos else 0)
            accs = [jnp.zeros((16,), jnp.float32)] * nacc
            for d in range(D):
                dcol = jnp.full((16,), d, jnp.int32)
                vcc = plsc.load_gather(vc_rows, [vc_idx, dcol])
                if with_pos:
                    voc = plsc.load_gather(vo_buf, [iota, dcol])
                    accs[0] = accs[0] + vcc * voc
                for j, k in enumerate(range(k_lo, k_hi)):
                    i = j + (1 if with_pos else 0)
                    vnc = plsc.load_gather(vng_buf, [iota_k + k, dcol])
                    accs[i] = accs[i] + vnc * vcc
            if with_pos:
                scores_v[0, pl.ds(col0, 16)] = accs[0]
            for j, k in enumerate(range(k_lo, k_hi)):
                i = j + (1 if with_pos else 0)
                scores_v[k + 1, pl.ds(col0, 16)] = -accs[i]
        for r in range(KNEG + 1, ROWS):
            scores_v[r, pl.ds(col0, 16)] = big

    @pl.loop(0, ng)
    def _(t):
        p = lax.rem(t, 2)
        vo_buf, vng_buf = vo_g.at[p], vng.at[p]
        sem = sem_a.at[p]
        _wait_group(out_hbm, vo_buf, vng_buf, sem)
        _compute_group(t, vo_buf, vng_buf)

        @pl.when(t < ng - 2)
        def _():
            _fire_group(t + 2, out_hbm, idx_o, idx_n, vo_buf, vng_buf, sem)

    pltpu.sync_copy(scores_v, scores_hbm.at[:, pl.ds(base, bpw)])


def _tc_loss_body(s_ref, o_ref):
    x = s_ref[...]
    ls = jnp.minimum(x, 0.0) - jnp.log1p(jnp.exp(-jnp.abs(x)))
    row = lax.broadcasted_iota(jnp.int32, x.shape, 0)
    ls = jnp.where(row < KNEG + 1, ls, 0.0)
    o_ref[0, 0] = -jnp.sum(ls) / s_ref.shape[1]


def kernel(center, context, negatives, in_embed, out_embed):
    b = center.shape[0]
    bpw = b // NW
    negflat = negatives.reshape(-1)

    one = (jnp.min(center) >> 31).astype(jnp.float32) + 1.0
    in_lin = in_embed * one

    scores = pl.kernel(
        _sc_scores_body,
        out_type=jax.ShapeDtypeStruct((ROWS, b), jnp.float32),
        mesh=plsc.VectorSubcoreMesh(core_axis_name="c", subcore_axis_name="s"),
        compiler_params=pltpu.CompilerParams(
            needs_layout_passes=False, use_tc_tiling_on_sc=False),
        scratch_types=[
            pltpu.VMEM((bpw,), jnp.int32),
            pltpu.VMEM((bpw,), jnp.int32),
            pltpu.VMEM((bpw * KNEG,), jnp.int32),
            pltpu.VMEM((bpw, D), jnp.float32),
            pltpu.VMEM((2, GSZ, D), jnp.float32),
            pltpu.VMEM((2, GSZ * KNEG, D), jnp.float32),
            pltpu.VMEM((ROWS, bpw), jnp.float32),
            pltpu.SemaphoreType.DMA,
            pltpu.SemaphoreType.DMA((2,)),
        ],
    )(in_lin, out_embed, center, context, negflat)

    loss = pl.pallas_call(
        _tc_loss_body,
        out_shape=jax.ShapeDtypeStruct((1, 1), jnp.float32),
        in_specs=[pl.BlockSpec((ROWS, b), lambda: (0, 0))],
        out_specs=pl.BlockSpec(memory_space=pltpu.SMEM),
    )(scores)
    return loss[0, 0]


# speedup vs baseline: 1.2394x; 1.2394x over previous
import jax
import jax.numpy as jnp
from jax import lax
from jax.experimental import pallas as pl
from jax.experimental.pallas import tpu as pltpu
from jax.experimental.pallas import tpu_sc as plsc

D = 64
KNEG = 20
NC, NS = 2, 16
NW = NC * NS
ROWS = 24
GSZ = 16
GN = GSZ * KNEG


def _fire_group(g, in2, out2, idx_cr, idx_or, idx_nr,
                vc_buf, vo_buf, vng_buf, sem):
    col0 = pl.multiple_of(g * GSZ, 8)
    nbase = pl.multiple_of(g * GN, 8)
    pltpu.async_copy(in2.at[idx_cr.at[pl.ds(col0, GSZ)]], vc_buf, sem)
    pltpu.async_copy(out2.at[idx_or.at[pl.ds(col0, GSZ)]], vo_buf, sem)
    pltpu.async_copy(out2.at[idx_nr.at[pl.ds(nbase, 128)]],
                     vng_buf.at[pl.ds(0, 128), :], sem)
    pltpu.async_copy(out2.at[idx_nr.at[pl.ds(nbase + 128, 128)]],
                     vng_buf.at[pl.ds(128, 128), :], sem)
    pltpu.async_copy(out2.at[idx_nr.at[pl.ds(nbase + 256, 64)]],
                     vng_buf.at[pl.ds(256, 64), :], sem)


def _wait_group(in2, vc_buf, vo_buf, vng_buf, sem):
    pltpu.make_async_copy(in2.at[pl.ds(0, GSZ)], vc_buf, sem).wait()
    pltpu.make_async_copy(in2.at[pl.ds(0, GSZ)], vo_buf, sem).wait()
    pltpu.make_async_copy(in2.at[pl.ds(0, 128)],
                          vng_buf.at[pl.ds(0, 128), :], sem).wait()
    pltpu.make_async_copy(in2.at[pl.ds(0, 128)],
                          vng_buf.at[pl.ds(128, 128), :], sem).wait()
    pltpu.make_async_copy(in2.at[pl.ds(0, 64)],
                          vng_buf.at[pl.ds(256, 64), :], sem).wait()


def _sc_scores_body(in2, out2, cen_hbm, ctx_hbm, neg_hbm, scores_hbm,
                    idx_cr, idx_or, idx_nr, par_c, par_o, par_n,
                    vc_g, vo_g, vng, scores_v, sem_a):
    bpw = par_c.shape[0]
    ng = bpw // GSZ
    wid = lax.axis_index("s") * NC + lax.axis_index("c")
    base = wid * bpw

    pltpu.sync_copy(cen_hbm.at[pl.ds(base, bpw)], idx_cr)
    pltpu.sync_copy(ctx_hbm.at[pl.ds(base, bpw)], idx_or)
    pltpu.sync_copy(neg_hbm.at[pl.ds(base * KNEG, bpw * KNEG)], idx_nr)

    @pl.loop(0, bpw // 16)
    def _(i):
        off = pl.multiple_of(i * 16, 8)
        v = idx_cr[pl.ds(off, 16)]
        idx_cr[pl.ds(off, 16)] = lax.shift_right_logical(v, 1)
        par_c[pl.ds(off, 16)] = lax.shift_left(lax.bitwise_and(v, 1), 6)
        w = idx_or[pl.ds(off, 16)]
        idx_or[pl.ds(off, 16)] = lax.shift_right_logical(w, 1)
        par_o[pl.ds(off, 16)] = lax.shift_left(lax.bitwise_and(w, 1), 6)

    @pl.loop(0, bpw * KNEG // 16)
    def _(i):
        off = pl.multiple_of(i * 16, 8)
        v = idx_nr[pl.ds(off, 16)]
        idx_nr[pl.ds(off, 16)] = lax.shift_right_logical(v, 1)
        par_n[pl.ds(off, 16)] = lax.shift_left(lax.bitwise_and(v, 1), 6)

    _fire_group(0, in2, out2, idx_cr, idx_or, idx_nr,
                vc_g.at[0], vo_g.at[0], vng.at[0], sem_a.at[0])
    _fire_group(1, in2, out2, idx_cr, idx_or, idx_nr,
                vc_g.at[1], vo_g.at[1], vng.at[1], sem_a.at[1])

    iota = lax.iota(jnp.int32, 16)
    iota_k = iota * KNEG
    big = jnp.full((16,), 1e4, jnp.float32)

    def _compute_group(g, vc_buf, vo_buf, vng_buf):
        col0 = pl.multiple_of(g * GSZ, 8)
        pc = par_c[pl.ds(col0, 16)]
        po = par_o[pl.ds(col0, 16)]
        nbase = g * GN
        for k_lo, k_hi, with_pos in ((0, 6, True), (6, 13, False),
                                     (13, 20, False)):
            nacc = (k_hi - k_lo) + (1 if with_pos else 0)
            accs = [jnp.zeros((16,), jnp.float32)] * nacc
            pns = [plsc.load_gather(par_n, [iota_k + (nbase + k)])
                   for k in range(k_lo, k_hi)]
            for d in range(D):
                vcc = plsc.load_gather(vc_buf, [iota, pc + d])
                if with_pos:
                    voc = plsc.load_gather(vo_buf, [iota, po + d])
                    accs[0] = accs[0] + vcc * voc
                for j, k in enumerate(range(k_lo, k_hi)):
                    i = j + (1 if with_pos else 0)
                    vnc = plsc.load_gather(vng_buf,
                                           [iota_k + k, pns[j] + d])
                    accs[i] = accs[i] + vnc * vcc
            if with_pos:
                scores_v[0, pl.ds(col0, 16)] = accs[0]
            for j, k in enumerate(range(k_lo, k_hi)):
                i = j + (1 if with_pos else 0)
                scores_v[k + 1, pl.ds(col0, 16)] = -accs[i]
        for r in range(KNEG + 1, ROWS):
            scores_v[r, pl.ds(col0, 16)] = big

    @pl.loop(0, ng)
    def _(t):
        p = lax.rem(t, 2)
        vc_buf, vo_buf, vng_buf = vc_g.at[p], vo_g.at[p], vng.at[p]
        sem = sem_a.at[p]
        _wait_group(in2, vc_buf, vo_buf, vng_buf, sem)
        _compute_group(t, vc_buf, vo_buf, vng_buf)

        @pl.when(t < ng - 2)
        def _():
            _fire_group(t + 2, in2, out2, idx_cr, idx_or, idx_nr,
                        vc_buf, vo_buf, vng_buf, sem)

    pltpu.sync_copy(scores_v, scores_hbm.at[:, pl.ds(base, bpw)])


def _tc_loss_body(s_ref, o_ref):
    x = s_ref[...]
    ls = jnp.minimum(x, 0.0) - jnp.log1p(jnp.exp(-jnp.abs(x)))
    row = lax.broadcasted_iota(jnp.int32, x.shape, 0)
    ls = jnp.where(row < KNEG + 1, ls, 0.0)
    o_ref[0, 0] = -jnp.sum(ls) / s_ref.shape[1]


def kernel(center, context, negatives, in_embed, out_embed):
    b = center.shape[0]
    bpw = b // NW
    v = in_embed.shape[0]
    negflat = negatives.reshape(-1)
    in2 = in_embed.reshape(v // 2, 2 * D)
    out2 = out_embed.reshape(v // 2, 2 * D)

    scores = pl.kernel(
        _sc_scores_body,
        out_type=jax.ShapeDtypeStruct((ROWS, b), jnp.float32),
        mesh=plsc.VectorSubcoreMesh(core_axis_name="c", subcore_axis_name="s"),
        compiler_params=pltpu.CompilerParams(
            needs_layout_passes=False, use_tc_tiling_on_sc=True),
        scratch_types=[
            pltpu.VMEM((bpw,), jnp.int32),
            pltpu.VMEM((bpw,), jnp.int32),
            pltpu.VMEM((bpw * KNEG,), jnp.int32),
            pltpu.VMEM((bpw,), jnp.int32),
            pltpu.VMEM((bpw,), jnp.int32),
            pltpu.VMEM((bpw * KNEG,), jnp.int32),
            pltpu.VMEM((2, GSZ, 2 * D), jnp.float32),
            pltpu.VMEM((2, GSZ, 2 * D), jnp.float32),
            pltpu.VMEM((2, GN, 2 * D), jnp.float32),
            pltpu.VMEM((ROWS, bpw), jnp.float32),
            pltpu.SemaphoreType.DMA((2,)),
        ],
    )(in2, out2, center, context, negflat)

    loss = pl.pallas_call(
        _tc_loss_body,
        out_shape=jax.ShapeDtypeStruct((1, 1), jnp.float32),
        in_specs=[pl.BlockSpec((ROWS, b), lambda: (0, 0))],
        out_specs=pl.BlockSpec(memory_space=pltpu.SMEM),
    )(scores)
    return loss[0, 0]


# speedup vs baseline: 1.3095x; 1.0566x over previous
import jax
import jax.numpy as jnp
from jax import lax
from jax.experimental import pallas as pl
from jax.experimental.pallas import tpu as pltpu
from jax.experimental.pallas import tpu_sc as plsc

D = 64
DP = 128
KNEG = 20
NC, NS = 2, 16
NW = NC * NS
ROWS = 24
GSZ = 16
GN = GSZ * KNEG


def _fire_group(g, in2, out2, idx_c, idx_o, idx_n,
                vc_buf, vo_buf, vng_buf, sem):
    col0 = pl.multiple_of(g * GSZ, 8)
    nbase = pl.multiple_of(g * GN, 8)
    pltpu.async_copy(in2.at[idx_c.at[pl.ds(col0, GSZ)]], vc_buf, sem)
    pltpu.async_copy(out2.at[idx_o.at[pl.ds(col0, GSZ)]], vo_buf, sem)
    pltpu.async_copy(out2.at[idx_n.at[pl.ds(nbase, 128)]],
                     vng_buf.at[pl.ds(0, 128), :], sem)
    pltpu.async_copy(out2.at[idx_n.at[pl.ds(nbase + 128, 128)]],
                     vng_buf.at[pl.ds(128, 128), :], sem)
    pltpu.async_copy(out2.at[idx_n.at[pl.ds(nbase + 256, 64)]],
                     vng_buf.at[pl.ds(256, 64), :], sem)


def _wait_group(in2, vc_buf, vo_buf, vng_buf, sem):
    pltpu.make_async_copy(in2.at[pl.ds(0, GSZ)], vc_buf, sem).wait()
    pltpu.make_async_copy(in2.at[pl.ds(0, GSZ)], vo_buf, sem).wait()
    pltpu.make_async_copy(in2.at[pl.ds(0, 128)],
                          vng_buf.at[pl.ds(0, 128), :], sem).wait()
    pltpu.make_async_copy(in2.at[pl.ds(0, 128)],
                          vng_buf.at[pl.ds(128, 128), :], sem).wait()
    pltpu.make_async_copy(in2.at[pl.ds(0, 64)],
                          vng_buf.at[pl.ds(256, 64), :], sem).wait()


def _sc_scores_body(in2, out2, cen_hbm, ctx_hbm, neg_hbm, scores_hbm,
                    idx_c, idx_o, idx_n, vc_g, vo_g, vng, scores_v, sem_a):
    bpw = idx_c.shape[0]
    ng = bpw // GSZ
    wid = lax.axis_index("s") * NC + lax.axis_index("c")
    base = wid * bpw

    pltpu.sync_copy(cen_hbm.at[pl.ds(base, bpw)], idx_c)
    pltpu.sync_copy(ctx_hbm.at[pl.ds(base, bpw)], idx_o)
    pltpu.sync_copy(neg_hbm.at[pl.ds(base * KNEG, bpw * KNEG)], idx_n)

    _fire_group(0, in2, out2, idx_c, idx_o, idx_n,
                vc_g.at[0], vo_g.at[0], vng.at[0], sem_a.at[0])
    _fire_group(1, in2, out2, idx_c, idx_o, idx_n,
                vc_g.at[1], vo_g.at[1], vng.at[1], sem_a.at[1])

    iota = lax.iota(jnp.int32, 16)
    iota_k = iota * KNEG
    big = jnp.full((16,), 1e4, jnp.float32)

    def _compute_group(g, vc_buf, vo_buf, vng_buf):
        col0 = pl.multiple_of(g * GSZ, 8)
        for k_lo, k_hi, with_pos in ((0, 6, True), (6, 13, False),
                                     (13, 20, False)):
            nacc = (k_hi - k_lo) + (1 if with_pos else 0)
            accs = [jnp.zeros((16,), jnp.float32)] * nacc
            for d in range(D):
                dcol = jnp.full((16,), d, jnp.int32)
                vcc = plsc.load_gather(vc_buf, [iota, dcol])
                if with_pos:
                    voc = plsc.load_gather(vo_buf, [iota, dcol])
                    accs[0] = accs[0] + vcc * voc
                for j, k in enumerate(range(k_lo, k_hi)):
                    i = j + (1 if with_pos else 0)
                    vnc = plsc.load_gather(vng_buf, [iota_k + k, dcol])
                    accs[i] = accs[i] + vnc * vcc
            if with_pos:
                scores_v[0, pl.ds(col0, 16)] = accs[0]
            for j, k in enumerate(range(k_lo, k_hi)):
                i = j + (1 if with_pos else 0)
                scores_v[k + 1, pl.ds(col0, 16)] = -accs[i]
        for r in range(KNEG + 1, ROWS):
            scores_v[r, pl.ds(col0, 16)] = big

    @pl.loop(0, ng)
    def _(t):
        p = lax.rem(t, 2)
        vc_buf, vo_buf, vng_buf = vc_g.at[p], vo_g.at[p], vng.at[p]
        sem = sem_a.at[p]
        _wait_group(in2, vc_buf, vo_buf, vng_buf, sem)
        _compute_group(t, vc_buf, vo_buf, vng_buf)

        @pl.when(t < ng - 2)
        def _():
            _fire_group(t + 2, in2, out2, idx_c, idx_o, idx_n,
                        vc_buf, vo_buf, vng_buf, sem)

    pltpu.sync_copy(scores_v, scores_hbm.at[:, pl.ds(base, bpw)])


def _tc_loss_body(s_ref, o_ref):
    x = s_ref[...]
    ls = jnp.minimum(x, 0.0) - jnp.log1p(jnp.exp(-jnp.abs(x)))
    row = lax.broadcasted_iota(jnp.int32, x.shape, 0)
    ls = jnp.where(row < KNEG + 1, ls, 0.0)
    o_ref[0, 0] = -jnp.sum(ls) / s_ref.shape[1]


def kernel(center, context, negatives, in_embed, out_embed):
    b = center.shape[0]
    bpw = b // NW
    negflat = negatives.reshape(-1)
    in2 = jnp.pad(in_embed, ((0, 0), (0, DP - D)))
    out2 = jnp.pad(out_embed, ((0, 0), (0, DP - D)))

    scores = pl.kernel(
        _sc_scores_body,
        out_type=jax.ShapeDtypeStruct((ROWS, b), jnp.float32),
        mesh=plsc.VectorSubcoreMesh(core_axis_name="c", subcore_axis_name="s"),
        compiler_params=pltpu.CompilerParams(
            needs_layout_passes=False, use_tc_tiling_on_sc=True),
        scratch_types=[
            pltpu.VMEM((bpw,), jnp.int32),
            pltpu.VMEM((bpw,), jnp.int32),
            pltpu.VMEM((bpw * KNEG,), jnp.int32),
            pltpu.VMEM((2, GSZ, DP), jnp.float32),
            pltpu.VMEM((2, GSZ, DP), jnp.float32),
            pltpu.VMEM((2, GN, DP), jnp.float32),
            pltpu.VMEM((ROWS, bpw), jnp.float32),
            pltpu.SemaphoreType.DMA((2,)),
        ],
    )(in2, out2, center, context, negflat)

    loss = pl.pallas_call(
        _tc_loss_body,
        out_shape=jax.ShapeDtypeStruct((1, 1), jnp.float32),
        in_specs=[pl.BlockSpec((ROWS, b), lambda: (0, 0))],
        out_specs=pl.BlockSpec(memory_space=pltpu.SMEM),
    )(scores)
    return loss[0, 0]
